# Initial kernel scaffold; baseline (speedup 1.0000x reference)
import jax
import jax.numpy as jnp
from jax.experimental import pallas as pl


def kernel(x, edge_index, W1, b1, Wl, bl, Wr, Wn):
    raise NotImplementedError("write your pallas kernel here")


# speedup vs baseline: 3.4812x; 3.4812x over previous
import functools

import jax
import jax.numpy as jnp
from jax import lax
from jax.experimental import pallas as pl
from jax.experimental.pallas import tpu as pltpu
from jax.experimental.pallas import tpu_sc as plsc

_N = 10000
_E = 320000
_D = 128
_H = 128
_C = 64

_NC = 2
_NS = 16
_EPT = _E // _NS
_K = 80
_NCH = _EPT // _K
_NP = 10240
_RPT = _NP // _NS


def _tc1_body(x_ref, w1_ref, b1_ref, feat_ref, hr_ref):
    h = jnp.dot(x_ref[...], w1_ref[...], preferred_element_type=jnp.float32)
    h = h + b1_ref[...]
    feat_ref[...] = h
    hr_ref[...] = jnp.maximum(h, 0.0)


def _stage1(x, W1, b1):
    blk = 1000
    return pl.pallas_call(
        _tc1_body,
        grid=(_N // blk,),
        in_specs=[
            pl.BlockSpec((blk, _D), lambda i: (i, 0)),
            pl.BlockSpec((_D, _H), lambda i: (0, 0)),
            pl.BlockSpec((1, _H), lambda i: (0, 0)),
        ],
        out_specs=[
            pl.BlockSpec((blk, _H), lambda i: (i, 0)),
            pl.BlockSpec((blk, _H), lambda i: (i, 0)),
        ],
        out_shape=[
            jax.ShapeDtypeStruct((_N, _H), jnp.float32),
            jax.ShapeDtypeStruct((_N, _H), jnp.float32),
        ],
    )(x, W1, b1.reshape(1, _H))


def _sc_body(hr_hbm, src_hbm, dst_hbm,
             parts_hbm, cnts_hbm,
             idxs_v, idxd_v, rows_v, acc_s, sem):
    cid = lax.axis_index("c")
    tid = lax.axis_index("s")

    z16 = jnp.zeros((16,), jnp.float32)

    def zrow(i, _):
        for j in range(_H // 16):
            rows_v[i, pl.ds(j * 16, 16)] = z16
        return ()
    lax.fori_loop(0, _K, zrow, ())

    def zacc(j, _):
        pltpu.sync_copy(rows_v, acc_s.at[pl.ds(tid * _RPT + j * _K, _K)])
        return ()
    lax.fori_loop(0, _RPT // _K, zacc, ())

    @pl.when(cid == 1)
    def _fill_ones():
        one16 = jnp.ones((16,), jnp.float32)

        def wones(i, _):
            for j in range(_H // 16):
                rows_v[i, pl.ds(j * 16, 16)] = one16
            return ()
        lax.fori_loop(0, _K, wones, ())

    plsc.subcore_barrier()

    def chunk_rows(c, _):
        base = tid * _EPT + c * _K
        pltpu.sync_copy(src_hbm.at[pl.ds(base, _K)], idxs_v)
        pltpu.sync_copy(dst_hbm.at[pl.ds(base, _K)], idxd_v)
        pltpu.async_copy(hr_hbm.at[idxs_v], rows_v, sem).wait()
        pltpu.sync_copy(rows_v, acc_s.at[idxd_v], add=True)
        return ()

    def chunk_cnts(c, _):
        base = tid * _EPT + c * _K
        pltpu.sync_copy(dst_hbm.at[pl.ds(base, _K)], idxd_v)
        pltpu.sync_copy(rows_v, acc_s.at[idxd_v], add=True)
        return ()

    @pl.when(cid == 0)
    def _rows():
        lax.fori_loop(0, _NCH, chunk_rows, ())

    @pl.when(cid == 1)
    def _cnts():
        lax.fori_loop(0, _NCH, chunk_cnts, ())

    plsc.subcore_barrier()

    def wacc(j, _):
        pltpu.sync_copy(acc_s.at[pl.ds(tid * _RPT + j * _K, _K)], rows_v)

        @pl.when(cid == 0)
        def _wp():
            pltpu.sync_copy(rows_v,
                            parts_hbm.at[pl.ds(tid * _RPT + j * _K, _K)])

        @pl.when(cid == 1)
        def _wc():
            pltpu.sync_copy(rows_v,
                            cnts_hbm.at[pl.ds(tid * _RPT + j * _K, _K)])
        return ()
    lax.fori_loop(0, _RPT // _K, wacc, ())


def _stage2(hr, src, dst):
    mesh = plsc.VectorSubcoreMesh(core_axis_name="c", subcore_axis_name="s")
    k = functools.partial(
        pl.kernel,
        mesh=mesh,
        out_type=[
            jax.ShapeDtypeStruct((_NP, _H), jnp.float32),
            jax.ShapeDtypeStruct((_NP, _H), jnp.float32),
        ],
        scratch_types=[
            pltpu.VMEM((_K,), jnp.int32),
            pltpu.VMEM((_K,), jnp.int32),
            pltpu.VMEM((_K, _H), jnp.float32),
            pltpu.VMEM_SHARED((_NP, _H), jnp.float32),
            pltpu.SemaphoreType.DMA,
        ],
    )(_sc_body)
    return k(hr, src, dst)


def _tc2_body(agg_ref, cnt_ref, hr_ref,
              wl_ref, bl_ref, wr_ref, wn_ref, out_ref):
    cnt = jnp.maximum(cnt_ref[:, :1], 1.0)
    agg = agg_ref[...] / cnt
    hr = hr_ref[...]
    h2 = (jnp.dot(agg, wl_ref[...], preferred_element_type=jnp.float32)
          + bl_ref[...]
          + jnp.dot(hr, wr_ref[...], preferred_element_type=jnp.float32))
    nrm = jnp.sqrt(jnp.sum(h2 * h2, axis=1, keepdims=True))
    xn = h2 / jnp.maximum(nrm, 1e-12)
    wn = wn_ref[...]
    wnn = wn / jnp.maximum(jnp.sqrt(jnp.sum(wn * wn, axis=0, keepdims=True)),
                           1e-12)
    out_ref[...] = 10.0 * jnp.dot(xn, wnn, preferred_element_type=jnp.float32)


def _stage3(parts, cnts, hr, Wl, bl, Wr, Wn):
    blk = 1000
    return pl.pallas_call(
        _tc2_body,
        grid=(_N // blk,),
        in_specs=[
            pl.BlockSpec((blk, _H), lambda i: (i, 0)),
            pl.BlockSpec((blk, _H), lambda i: (i, 0)),
            pl.BlockSpec((blk, _H), lambda i: (i, 0)),
            pl.BlockSpec((_H, _H), lambda i: (0, 0)),
            pl.BlockSpec((1, _H), lambda i: (0, 0)),
            pl.BlockSpec((_H, _H), lambda i: (0, 0)),
            pl.BlockSpec((_H, _C), lambda i: (0, 0)),
        ],
        out_specs=pl.BlockSpec((blk, _C), lambda i: (i, 0)),
        out_shape=jax.ShapeDtypeStruct((_N, _C), jnp.float32),
    )(parts, cnts, hr, Wl, bl.reshape(1, _H), Wr, Wn)


@jax.jit
def kernel(x, edge_index, W1, b1, Wl, bl, Wr, Wn):
    feat, hr = _stage1(x, W1, b1)
    src = edge_index[0]
    dst = edge_index[1]
    parts, cnts = _stage2(hr, src, dst)
    out = _stage3(parts, cnts, hr, Wl, bl, Wr, Wn)
    return (feat, out)


# speedup vs baseline: 5.4018x; 1.5517x over previous
import functools

import jax
import jax.numpy as jnp
from jax import lax
from jax.experimental import pallas as pl
from jax.experimental.pallas import tpu as pltpu
from jax.experimental.pallas import tpu_sc as plsc

_N = 10000
_E = 320000
_D = 128
_H = 128
_C = 64

_NC = 2
_NS = 16
_EPT = _E // _NS
_K = 40
_U = 4
_NB = _EPT // (_K * _U)
_NP = 10240
_RPT = _NP // _NS


def _tc1_body(x_ref, w1_ref, b1_ref, feat_ref, hr_ref):
    h = jnp.dot(x_ref[...], w1_ref[...], preferred_element_type=jnp.float32)
    h = h + b1_ref[...]
    feat_ref[...] = h
    hr_ref[...] = jnp.maximum(h, 0.0)


def _stage1(x, W1, b1):
    blk = 1000
    return pl.pallas_call(
        _tc1_body,
        grid=(_N // blk,),
        in_specs=[
            pl.BlockSpec((blk, _D), lambda i: (i, 0)),
            pl.BlockSpec((_D, _H), lambda i: (0, 0)),
            pl.BlockSpec((1, _H), lambda i: (0, 0)),
        ],
        out_specs=[
            pl.BlockSpec((blk, _H), lambda i: (i, 0)),
            pl.BlockSpec((blk, _H), lambda i: (i, 0)),
        ],
        out_shape=[
            jax.ShapeDtypeStruct((_N, _H), jnp.float32),
            jax.ShapeDtypeStruct((_N, _H), jnp.float32),
        ],
    )(x, W1, b1.reshape(1, _H))


def _sc_body(hr_hbm, src_hbm, dst_hbm,
             parts_hbm, cnts_hbm,
             *refs):
    idxs = refs[0:_U]
    idxd = refs[_U:2 * _U]
    rows = refs[2 * _U:3 * _U]
    acc_s = refs[3 * _U]
    semi, semg, sems = refs[3 * _U + 1:3 * _U + 4]

    cid = lax.axis_index("c")
    tid = lax.axis_index("s")

    z16 = jnp.zeros((16,), jnp.float32)

    def zrow(i, _):
        for j in range(_H // 16):
            rows[0][i, pl.ds(j * 16, 16)] = z16
        return ()
    lax.fori_loop(0, _K, zrow, ())

    def zacc(j, _):
        pltpu.sync_copy(rows[0], acc_s.at[pl.ds(tid * _RPT + j * _K, _K)])
        return ()
    lax.fori_loop(0, _RPT // _K, zacc, ())

    @pl.when(cid == 1)
    def _fill_ones():
        one16 = jnp.ones((16,), jnp.float32)

        def wones(i, _):
            for j in range(_H // 16):
                rows[0][i, pl.ds(j * 16, 16)] = one16
            return ()
        lax.fori_loop(0, _K, wones, ())

    plsc.subcore_barrier()

    def body_rows(s, _):
        base0 = tid * _EPT + s * (_U * _K)
        di = []
        for u in range(_U):
            di.append(pltpu.async_copy(
                src_hbm.at[pl.ds(base0 + u * _K, _K)], idxs[u], semi))
            di.append(pltpu.async_copy(
                dst_hbm.at[pl.ds(base0 + u * _K, _K)], idxd[u], semi))
        for d in di:
            d.wait()
        dg = [pltpu.async_copy(hr_hbm.at[idxs[u]], rows[u], semg)
              for u in range(_U)]
        for d in dg:
            d.wait()
        ds = [pltpu.async_copy(rows[u], acc_s.at[idxd[u]], sems, add=True)
              for u in range(_U)]
        for d in ds:
            d.wait()
        return ()

    def body_cnts(s, _):
        base0 = tid * _EPT + s * (_U * _K)
        di = [pltpu.async_copy(
            dst_hbm.at[pl.ds(base0 + u * _K, _K)], idxd[u], semi)
            for u in range(_U)]
        for d in di:
            d.wait()
        ds = [pltpu.async_copy(rows[0], acc_s.at[idxd[u]], sems, add=True)
              for u in range(_U)]
        for d in ds:
            d.wait()
        return ()

    @pl.when(cid == 0)
    def _rows():
        lax.fori_loop(0, _NB, body_rows, ())

    @pl.when(cid == 1)
    def _cnts():
        lax.fori_loop(0, _NB, body_cnts, ())

    plsc.subcore_barrier()

    def wacc(j, _):
        pltpu.sync_copy(acc_s.at[pl.ds(tid * _RPT + j * _K, _K)], rows[0])

        @pl.when(cid == 0)
        def _wp():
            pltpu.sync_copy(rows[0],
                            parts_hbm.at[pl.ds(tid * _RPT + j * _K, _K)])

        @pl.when(cid == 1)
        def _wc():
            pltpu.sync_copy(rows[0],
                            cnts_hbm.at[pl.ds(tid * _RPT + j * _K, _K)])
        return ()
    lax.fori_loop(0, _RPT // _K, wacc, ())


def _stage2(hr, src, dst):
    mesh = plsc.VectorSubcoreMesh(core_axis_name="c", subcore_axis_name="s")
    k = functools.partial(
        pl.kernel,
        mesh=mesh,
        out_type=[
            jax.ShapeDtypeStruct((_NP, _H), jnp.float32),
            jax.ShapeDtypeStruct((_NP, _H), jnp.float32),
        ],
        scratch_types=(
            [pltpu.VMEM((_K,), jnp.int32) for _ in range(2 * _U)]
            + [pltpu.VMEM((_K, _H), jnp.float32) for _ in range(_U)]
            + [pltpu.VMEM_SHARED((_NP, _H), jnp.float32)]
            + [pltpu.SemaphoreType.DMA for _ in range(3)]
        ),
    )(_sc_body)
    return k(hr, src, dst)


def _tc2_body(agg_ref, cnt_ref, hr_ref,
              wl_ref, bl_ref, wr_ref, wn_ref, out_ref):
    cnt = jnp.maximum(cnt_ref[:, :1], 1.0)
    agg = agg_ref[...] / cnt
    hr = hr_ref[...]
    h2 = (jnp.dot(agg, wl_ref[...], preferred_element_type=jnp.float32)
          + bl_ref[...]
          + jnp.dot(hr, wr_ref[...], preferred_element_type=jnp.float32))
    nrm = jnp.sqrt(jnp.sum(h2 * h2, axis=1, keepdims=True))
    xn = h2 / jnp.maximum(nrm, 1e-12)
    wn = wn_ref[...]
    wnn = wn / jnp.maximum(jnp.sqrt(jnp.sum(wn * wn, axis=0, keepdims=True)),
                           1e-12)
    out_ref[...] = 10.0 * jnp.dot(xn, wnn, preferred_element_type=jnp.float32)


def _stage3(parts, cnts, hr, Wl, bl, Wr, Wn):
    blk = 1000
    return pl.pallas_call(
        _tc2_body,
        grid=(_N // blk,),
        in_specs=[
            pl.BlockSpec((blk, _H), lambda i: (i, 0)),
            pl.BlockSpec((blk, _H), lambda i: (i, 0)),
            pl.BlockSpec((blk, _H), lambda i: (i, 0)),
            pl.BlockSpec((_H, _H), lambda i: (0, 0)),
            pl.BlockSpec((1, _H), lambda i: (0, 0)),
            pl.BlockSpec((_H, _H), lambda i: (0, 0)),
            pl.BlockSpec((_H, _C), lambda i: (0, 0)),
        ],
        out_specs=pl.BlockSpec((blk, _C), lambda i: (i, 0)),
        out_shape=jax.ShapeDtypeStruct((_N, _C), jnp.float32),
    )(parts, cnts, hr, Wl, bl.reshape(1, _H), Wr, Wn)


@jax.jit
def kernel(x, edge_index, W1, b1, Wl, bl, Wr, Wn):
    feat, hr = _stage1(x, W1, b1)
    src = edge_index[0]
    dst = edge_index[1]
    parts, cnts = _stage2(hr, src, dst)
    out = _stage3(parts, cnts, hr, Wl, bl, Wr, Wn)
    return (feat, out)
